# baseline (speedup 1.0000x reference)
import functools
import math

import jax
from jax import lax
import jax.numpy as jnp
from jax.experimental import pallas as pl
from jax.experimental.pallas import tpu as pltpu
from jax.experimental.pallas import tpu_sc as plsc

_SMOOTHING = 0.1
_CONF = 1.0 - _SMOOTHING
_BLK_R = 16


def _dense_kernel(a_ref, b_ref, out_ref):
    @pl.when(pl.program_id(0) == 0)
    def _init():
        out_ref[0, 0] = 0.0

    out_ref[0, 0] += jnp.sum(a_ref[...]) + jnp.sum(b_ref[...])


def _sc_gather_body(pred_hbm, gold_hbm, out_hbm, gold_v, win_v, p0win_v,
                    acc_v, sem, *, V, b_per_w, n_sub, NC):
    s = _SMOOTHING / (V - 2)
    slogs = s * math.log(s)
    wid = lax.axis_index("s") * NC + lax.axis_index("c")
    base = wid * b_per_w
    pltpu.sync_copy(gold_hbm.at[pl.ds(base, b_per_w)], gold_v)
    iota16 = lax.iota(jnp.int32, 16)
    copies = []
    for j in range(n_sub):
        g = gold_v[pl.ds(j * 16, 16)]
        start_vec = lax.bitwise_and(g, ~15)
        for i in range(16):
            r = j * 16 + i
            start = pl.multiple_of(start_vec[i], 16)
            copies.append(pltpu.async_copy(
                pred_hbm.at[base + r, pl.ds(start, 16)], win_v.at[r], sem))
            copies.append(pltpu.async_copy(
                pred_hbm.at[base + r, pl.ds(0, 16)], p0win_v.at[r], sem))
    for cp in copies:
        cp.wait()
    acc = jnp.zeros((16,), jnp.float32)
    mask0 = jnp.where(iota16 == 0, 1.0, 0.0).astype(jnp.float32)
    for j in range(n_sub):
        g = gold_v[pl.ds(j * 16, 16)]
        lane_vec = lax.bitwise_and(g, 15)
        for i in range(16):
            r = j * 16 + i
            pgv = jnp.where(iota16 == lane_vec[i], win_v[r], 0.0)
            p0v = p0win_v[r] * mask0
            gzf = jnp.where(g[i] == 0, 1.0, 0.0).astype(jnp.float32)
            acc = (acc + s * p0v + (s - _CONF) * pgv
                   + gzf * (slogs * mask0 - s * p0v))
    acc_v[...] = acc
    pltpu.sync_copy(acc_v, out_hbm.at[wid])


def kernel(pred, gold):
    B, V = pred.shape
    s = _SMOOTHING / (V - 2)
    k1 = (V - 2) * s * math.log(s) + _CONF * math.log(_CONF)

    blk_r = _BLK_R
    n_steps = B // (2 * blk_r)
    dense = pl.pallas_call(
        _dense_kernel,
        grid=(n_steps,),
        in_specs=[
            pl.BlockSpec((blk_r, V), lambda i: (2 * i, 0)),
            pl.BlockSpec((blk_r, V), lambda i: (2 * i + 1, 0)),
        ],
        out_specs=pl.BlockSpec(memory_space=pltpu.SMEM),
        out_shape=jax.ShapeDtypeStruct((1, 1), jnp.float32),
        compiler_params=pltpu.CompilerParams(
            dimension_semantics=("arbitrary",),
        ),
    )(pred, pred)

    info = plsc.get_sparse_core_info()
    NC, NS = info.num_cores, info.num_subcores
    NW = NC * NS
    b_per_w = B // NW
    n_sub = b_per_w // 16
    sc_fn = functools.partial(
        pl.kernel,
        mesh=plsc.VectorSubcoreMesh(core_axis_name="c", subcore_axis_name="s"),
        out_type=jax.ShapeDtypeStruct((NW, 16), jnp.float32),
        scratch_types=[
            pltpu.VMEM((b_per_w,), jnp.int32),
            pltpu.VMEM((b_per_w, 16), jnp.float32),
            pltpu.VMEM((b_per_w, 16), jnp.float32),
            pltpu.VMEM((16,), jnp.float32),
            pltpu.SemaphoreType.DMA,
        ],
        compiler_params=pltpu.CompilerParams(use_tc_tiling_on_sc=False),
    )(functools.partial(_sc_gather_body, V=V, b_per_w=b_per_w,
                        n_sub=n_sub, NC=NC))
    sc_part = sc_fn(pred, gold)

    return B * k1 - s * dense[0, 0] + jnp.sum(sc_part)


# speedup vs baseline: 2.1564x; 2.1564x over previous
import functools
import math

import jax
from jax import lax
import jax.numpy as jnp
from jax.experimental import pallas as pl
from jax.experimental.pallas import tpu as pltpu
from jax.experimental.pallas import tpu_sc as plsc

_SMOOTHING = 0.1
_CONF = 1.0 - _SMOOTHING
_BLK_R = 16


def _dense_kernel(a_ref, b_ref, out_ref):
    @pl.when(pl.program_id(0) == 0)
    def _init():
        out_ref[0, 0] = 0.0

    out_ref[0, 0] += jnp.sum(a_ref[...]) + jnp.sum(b_ref[...])


def _sc_gather_body(pred_hbm, gold_hbm, out_hbm, gold_v, win_v, p0win_v,
                    acc_v, sem, *, V, b_per_w, n_sub, NC):
    s = _SMOOTHING / (V - 2)
    slogs = s * math.log(s)
    wid = lax.axis_index("s") * NC + lax.axis_index("c")
    base = wid * b_per_w
    pltpu.sync_copy(gold_hbm.at[pl.ds(base, b_per_w)], gold_v)
    iota16 = lax.iota(jnp.int32, 16)
    copies = []
    for j in range(n_sub):
        g = gold_v[pl.ds(j * 16, 16)]
        col0_vec = lax.bitwise_and(g, ~127)
        for i in range(16):
            r = j * 16 + i
            col0 = pl.multiple_of(col0_vec[i], 128)
            copies.append(pltpu.async_copy(
                pred_hbm.at[pl.ds(base + (r & ~7), 8),
                            pl.ds(col0, 128)],
                win_v.at[r], sem))
    for t in range(b_per_w // 8):
        copies.append(pltpu.async_copy(
            pred_hbm.at[pl.ds(base + 8 * t, 8), pl.ds(0, 128)],
            p0win_v.at[t], sem))
    for cp in copies:
        cp.wait()
    acc = jnp.zeros((16,), jnp.float32)
    mask0 = jnp.where(iota16 == 0, 1.0, 0.0).astype(jnp.float32)
    for j in range(n_sub):
        g = gold_v[pl.ds(j * 16, 16)]
        lane_vec = lax.bitwise_and(g, 127)
        for i in range(16):
            r = j * 16 + i
            lane = lane_vec[i]
            for k in range(8):
                pgv = jnp.where(iota16 + 16 * k == lane,
                                win_v[r, r & 7, pl.ds(16 * k, 16)], 0.0)
                acc = acc + (s - _CONF) * pgv
            p0v = p0win_v[r // 8, r & 7, pl.ds(0, 16)] * mask0
            gzf = jnp.where(g[i] == 0, 1.0, 0.0).astype(jnp.float32)
            acc = acc + s * p0v + gzf * (slogs * mask0 - s * p0v)
    acc_v[...] = acc
    pltpu.sync_copy(acc_v, out_hbm.at[wid])


def kernel(pred, gold):
    B, V = pred.shape
    s = _SMOOTHING / (V - 2)
    k1 = (V - 2) * s * math.log(s) + _CONF * math.log(_CONF)

    blk_r = _BLK_R
    n_steps = B // (2 * blk_r)
    dense = pl.pallas_call(
        _dense_kernel,
        grid=(n_steps,),
        in_specs=[
            pl.BlockSpec((blk_r, V), lambda i: (2 * i, 0)),
            pl.BlockSpec((blk_r, V), lambda i: (2 * i + 1, 0)),
        ],
        out_specs=pl.BlockSpec(memory_space=pltpu.SMEM),
        out_shape=jax.ShapeDtypeStruct((1, 1), jnp.float32),
        compiler_params=pltpu.CompilerParams(
            dimension_semantics=("arbitrary",),
        ),
    )(pred, pred)

    info = plsc.get_sparse_core_info()
    NC, NS = info.num_cores, info.num_subcores
    NW = NC * NS
    b_per_w = B // NW
    n_sub = b_per_w // 16
    sc_fn = functools.partial(
        pl.kernel,
        mesh=plsc.VectorSubcoreMesh(core_axis_name="c", subcore_axis_name="s"),
        out_type=jax.ShapeDtypeStruct((NW, 16), jnp.float32),
        scratch_types=[
            pltpu.VMEM((b_per_w,), jnp.int32),
            pltpu.VMEM((b_per_w, 8, 128), jnp.float32),
            pltpu.VMEM((b_per_w // 8, 8, 128), jnp.float32),
            pltpu.VMEM((16,), jnp.float32),
            pltpu.SemaphoreType.DMA,
        ],
    )(functools.partial(_sc_gather_body, V=V, b_per_w=b_per_w,
                        n_sub=n_sub, NC=NC))
    sc_part = sc_fn(pred, gold)

    return B * k1 - s * dense[0, 0] + jnp.sum(sc_part)
